# baseline (speedup 1.0000x reference)
import jax
import jax.numpy as jnp
from jax import lax
from jax.experimental import pallas as pl
from jax.experimental.pallas import tpu as pltpu
from jax.experimental.pallas import tpu_sc as plsc

B = 1024
F = 26
T = 26
D = 32
V = 1000 * F
NC = 2
NS = 16
NW = NC * NS

PAIRS = B * F
PAIRS_PER_W = PAIRS // NW
HALF = PAIRS_PER_W // 2


def _sc_body(tab_hbm, x_hbm, out_hbm, xv, bv, idx0, idx1, rows0, rows1,
             gsem, wsem0, wsem1):
    w = lax.axis_index("s") * NC + lax.axis_index("c")
    lane = lax.iota(jnp.int32, 16)

    pltpu.sync_copy(x_hbm.at[pl.ds(w * PAIRS_PER_W, PAIRS_PER_W)], xv)

    def bv_body(j, carry):
        p16 = j * 16 + lane
        f16 = lax.rem(p16, F)
        bv[pl.ds(j * 16, 16)] = xv[pl.ds(j * 16, 16)] + f16 * 1000
        return carry

    lax.fori_loop(0, PAIRS_PER_W // 16, bv_body, 0)

    idxs = (idx0, idx1)
    rows = (rows0, rows1)
    wsems = (wsem0, wsem1)
    p0 = w * PAIRS_PER_W

    for t in range(T):
        it = t % 2
        idxv = idxs[it]

        def idx_body(j, carry):
            idxv[pl.ds(j * 16, 16)] = (bv[pl.ds(j * 16, 16)] + t * V) * 4
            return carry

        lax.fori_loop(0, PAIRS_PER_W // 16, idx_body, 0)

        for h in range(2):
            if t >= 1:
                pltpu.make_async_copy(
                    rows[h],
                    out_hbm.at[pl.ds(0, HALF), 0, :],
                    wsems[h]).wait()
            pltpu.async_copy(
                tab_hbm.at[idxv.at[pl.ds(h * HALF, HALF)]],
                rows[h], gsem).wait()
            pltpu.async_copy(
                rows[h],
                out_hbm.at[pl.ds(p0 + h * HALF, HALF), t, :],
                wsems[h])

    for h in range(2):
        pltpu.make_async_copy(
            rows[h],
            out_hbm.at[pl.ds(0, HALF), 0, :],
            wsems[h]).wait()


@jax.jit
def _sc_gather(tab_pad, x_flat):
    mesh = plsc.VectorSubcoreMesh(
        core_axis_name="c", subcore_axis_name="s",
        num_cores=NC, num_subcores=NS)
    run = pl.kernel(
        _sc_body,
        out_type=jax.ShapeDtypeStruct((PAIRS, 32, D), jnp.float32),
        mesh=mesh,
        compiler_params=pltpu.CompilerParams(use_tc_tiling_on_sc=False),
        scratch_types=[
            pltpu.VMEM((PAIRS_PER_W,), jnp.int32),
            pltpu.VMEM((PAIRS_PER_W,), jnp.int32),
            pltpu.VMEM((PAIRS_PER_W,), jnp.int32),
            pltpu.VMEM((PAIRS_PER_W,), jnp.int32),
            pltpu.VMEM((HALF, D), jnp.float32),
            pltpu.VMEM((HALF, D), jnp.float32),
            pltpu.SemaphoreType.DMA,
            pltpu.SemaphoreType.DMA,
            pltpu.SemaphoreType.DMA,
        ],
    )
    return run(tab_pad, x_flat)


def kernel(x, tables):
    tab_pad = jnp.pad(
        tables.reshape(T * V, D), ((0, 0), (0, 128 - D))).reshape(4 * T * V, D)
    x_flat = x.reshape(PAIRS)
    out = _sc_gather(tab_pad, x_flat)
    return out[:, :T, :].reshape(B, F, T, D)


# speedup vs baseline: 1.5199x; 1.5199x over previous
import jax
import jax.numpy as jnp
from jax import lax
from jax.experimental import pallas as pl
from jax.experimental.pallas import tpu as pltpu
from jax.experimental.pallas import tpu_sc as plsc

B = 1024
F = 26
T = 26
D = 32
V = 1000 * F
NC = 2
NS = 16
NW = NC * NS

PAIRS = B * F
PAIRS_PER_W = PAIRS // NW
HALF = PAIRS_PER_W // 2


def _sc_body(tab_hbm, x_hbm, out_hbm, xv, bv, idx0, idx1, rows0, rows1,
             gsem, wsem0, wsem1):
    w = lax.axis_index("s") * NC + lax.axis_index("c")
    lane = lax.iota(jnp.int32, 16)

    pltpu.sync_copy(x_hbm.at[pl.ds(w * PAIRS_PER_W, PAIRS_PER_W)], xv)

    def bv_body(j, carry):
        p16 = j * 16 + lane
        f16 = lax.rem(p16, F)
        bv[pl.ds(j * 16, 16)] = xv[pl.ds(j * 16, 16)] + f16 * 1000
        return carry

    lax.fori_loop(0, PAIRS_PER_W // 16, bv_body, 0)

    idxs = (idx0, idx1)
    rows = (rows0, rows1)
    wsems = (wsem0, wsem1)
    p0 = w * PAIRS_PER_W

    for t in range(T):
        it = t % 2
        idxv = idxs[it]

        def idx_body(j, carry):
            idxv[pl.ds(j * 16, 16)] = (bv[pl.ds(j * 16, 16)] + t * V) * 4
            return carry

        lax.fori_loop(0, PAIRS_PER_W // 16, idx_body, 0)

        for h in range(2):
            if t >= 1:
                pltpu.make_async_copy(
                    rows[h],
                    out_hbm.at[pl.ds(0, HALF), 0, pl.ds(0, D)],
                    wsems[h]).wait()
            pltpu.async_copy(
                tab_hbm.at[idxv.at[pl.ds(h * HALF, HALF)]],
                rows[h], gsem).wait()
            pltpu.async_copy(
                rows[h],
                out_hbm.at[pl.ds(p0 + h * HALF, HALF), t, pl.ds(0, D)],
                wsems[h])

    for h in range(2):
        pltpu.make_async_copy(
            rows[h],
            out_hbm.at[pl.ds(0, HALF), 0, pl.ds(0, D)],
            wsems[h]).wait()


@jax.jit
def _sc_gather(tab_pad, x_flat):
    mesh = plsc.VectorSubcoreMesh(
        core_axis_name="c", subcore_axis_name="s",
        num_cores=NC, num_subcores=NS)
    run = pl.kernel(
        _sc_body,
        out_type=jax.ShapeDtypeStruct((PAIRS, 32, 128), jnp.float32),
        mesh=mesh,
        compiler_params=pltpu.CompilerParams(use_tc_tiling_on_sc=False),
        scratch_types=[
            pltpu.VMEM((PAIRS_PER_W,), jnp.int32),
            pltpu.VMEM((PAIRS_PER_W,), jnp.int32),
            pltpu.VMEM((PAIRS_PER_W,), jnp.int32),
            pltpu.VMEM((PAIRS_PER_W,), jnp.int32),
            pltpu.VMEM((HALF, D), jnp.float32),
            pltpu.VMEM((HALF, D), jnp.float32),
            pltpu.SemaphoreType.DMA,
            pltpu.SemaphoreType.DMA,
            pltpu.SemaphoreType.DMA,
        ],
    )
    return run(tab_pad, x_flat)


def kernel(x, tables):
    tab_pad = jnp.pad(
        tables.reshape(T * V, D), ((0, 0), (0, 128 - D))).reshape(4 * T * V, D)
    x_flat = x.reshape(PAIRS)
    out = _sc_gather(tab_pad, x_flat)
    return out[:, :T, :D].reshape(B, F, T, D)
